# baseline (speedup 1.0000x reference)
import functools

import jax
import jax.numpy as jnp
import numpy as np
from jax import lax
from jax.experimental import pallas as pl
from jax.experimental.pallas import tpu as pltpu
from jax.experimental.pallas import tpu_sc as plsc

N = 10000
E = 320000
D = 128
ROW = 136
NC = 2
NS = 16
NW = NC * NS
EPT = E // NW
B = 40
NBLK = EPT // B
IC = 10
NCHUNK = NBLK // IC
NROWCHUNK = (N + B - 1) // B


def _node_mm_body(x_ref, wl_ref, wr_ref, xl_ref, xr_ref):
    xb = x_ref[...]
    xl_ref[...] = jnp.dot(
        xb, wl_ref[...], preferred_element_type=jnp.float32
    ).astype(jnp.bfloat16)
    xr_ref[...] = jnp.dot(
        xb, wr_ref[...], preferred_element_type=jnp.float32
    ).astype(jnp.bfloat16)


_node_mm = pl.pallas_call(
    _node_mm_body,
    grid=(10,),
    in_specs=[
        pl.BlockSpec((1000, D), lambda i: (i, 0)),
        pl.BlockSpec((D, D), lambda i: (0, 0)),
        pl.BlockSpec((D, D), lambda i: (0, 0)),
    ],
    out_specs=[pl.BlockSpec((1000, D), lambda i: (i, 0))] * 2,
    out_shape=[jax.ShapeDtypeStruct((N, D), jnp.bfloat16)] * 2,
)


def _edge_mm_body(ea_ref, we_ref, xe_ref):
    xe_ref[...] = jnp.dot(ea_ref[...], we_ref[...],
                          preferred_element_type=jnp.float32).astype(jnp.bfloat16)


_edge_mm = pl.pallas_call(
    _edge_mm_body,
    grid=(80,),
    in_specs=[
        pl.BlockSpec((4000, 16), lambda i: (i, 0)),
        pl.BlockSpec((16, D), lambda i: (0, 0)),
    ],
    out_specs=pl.BlockSpec((4000, D), lambda i: (i, 0)),
    out_shape=jax.ShapeDtypeStruct((E, D), jnp.bfloat16),
)


def _combine_mm_body(acc_ref, b_ref, wl_ref, wr_ref, xl_ref, xr_ref):
    s = acc_ref[0] + acc_ref[1]
    num = s[:, :D]
    den = s[:, D:D + 1]
    h = num / (den + 1e-16) + b_ref[...]
    h = jnp.where(h > 0, h, jnp.exp(h) - 1.0)
    xl_ref[...] = jnp.dot(
        h, wl_ref[...], preferred_element_type=jnp.float32
    ).astype(jnp.bfloat16)
    xr_ref[...] = jnp.dot(
        h, wr_ref[...], preferred_element_type=jnp.float32
    ).astype(jnp.bfloat16)


_combine_mm = pl.pallas_call(
    _combine_mm_body,
    grid=(10,),
    in_specs=[
        pl.BlockSpec((2, 1000, ROW), lambda i: (0, i, 0)),
        pl.BlockSpec((1, D), lambda i: (0, 0)),
        pl.BlockSpec((D, D), lambda i: (0, 0)),
        pl.BlockSpec((D, D), lambda i: (0, 0)),
    ],
    out_specs=[pl.BlockSpec((1000, D), lambda i: (i, 0))] * 2,
    out_shape=[jax.ShapeDtypeStruct((N, D), jnp.bfloat16)] * 2,
)


def _combine_final_body(acc_ref, b_ref, out_ref):
    s = acc_ref[0] + acc_ref[1]
    num = s[:, :D]
    den = s[:, D:D + 1]
    out_ref[...] = num / (den + 1e-16) + b_ref[...]


_combine_final = pl.pallas_call(
    _combine_final_body,
    grid=(10,),
    in_specs=[
        pl.BlockSpec((2, 1000, ROW), lambda i: (0, i, 0)),
        pl.BlockSpec((1, D), lambda i: (0, 0)),
    ],
    out_specs=pl.BlockSpec((1000, D), lambda i: (i, 0)),
    out_shape=jax.ShapeDtypeStruct((N, D), jnp.float32),
)


def _edge_pass_body(src_h, dst_h, xl_h, xr_h, xe_h, att_h, acc_h,
                    idxs_v, idxd_v, xl_v0, xl_v1, xr_v0, xr_v1, xe_v0, xe_v1,
                    out_v0, out_v1, att_v, acc_sh,
                    gsem0, gsem1, ssem0, ssem1, isem):
    cid = lax.axis_index("c")
    sid = lax.axis_index("s")
    wid = cid * NS + sid
    xl_vs = (xl_v0, xl_v1)
    xr_vs = (xr_v0, xr_v1)
    xe_vs = (xe_v0, xe_v1)
    out_vs = (out_v0, out_v1)
    gsems = (gsem0, gsem1)
    ssems = (ssem0, ssem1)

    z = jnp.zeros((16,), jnp.float32)

    def zv_body(i, carry):
        out_v0[i, pl.ds(ROW - 16, 16)] = z
        for c in range(8):
            out_v0[i, pl.ds(c * 16, 16)] = z
        return carry

    lax.fori_loop(0, B, zv_body, 0)

    def zacc_body(k, carry):
        j = sid + k * NS

        @pl.when(j < NROWCHUNK)
        def _():
            pltpu.sync_copy(out_v0, acc_sh.at[pl.ds(j * B, B)])

        return carry

    lax.fori_loop(0, (NROWCHUNK + NS - 1) // NS, zacc_body, 0)

    pltpu.sync_copy(src_h.at[wid, pl.ds(0, IC)], idxs_v.at[0])
    pltpu.sync_copy(dst_h.at[wid, pl.ds(0, IC)], idxd_v.at[0])
    pltpu.sync_copy(att_h, att_v)
    plsc.subcore_barrier()

    attc = [att_v[pl.ds(c * 16, 16)] for c in range(8)]
    shift16 = jnp.full((16,), 16, jnp.int32)
    himask = jnp.full((16,), -0x10000, jnp.int32)

    def _lo(v):
        return lax.bitcast_convert_type(lax.shift_left(v, shift16), jnp.float32)

    def _hi(v):
        return lax.bitcast_convert_type(jnp.bitwise_and(v, himask), jnp.float32)

    def _lanesum(v):
        return lax.broadcast(jnp.sum(v), (16,))

    def _idx(ref, blk):
        return ref.at[(blk // IC) % 2, blk % IC]

    def issue_idx_chunk(c):
        pltpu.async_copy(src_h.at[wid, pl.ds(c * IC, IC)], idxs_v.at[c % 2], isem)
        pltpu.async_copy(dst_h.at[wid, pl.ds(c * IC, IC)], idxd_v.at[c % 2], isem)

    def wait_idx_chunk(c):
        pltpu.make_async_copy(
            src_h.at[wid, pl.ds(c * IC, IC)], idxs_v.at[c % 2], isem).wait()
        pltpu.make_async_copy(
            dst_h.at[wid, pl.ds(c * IC, IC)], idxd_v.at[c % 2], isem).wait()

    def issue_gathers(b, blk):
        base = pl.multiple_of(wid * EPT, 8) + blk * B
        pltpu.async_copy(xl_h.at[_idx(idxs_v, blk)], xl_vs[b], gsems[b])
        pltpu.async_copy(xr_h.at[_idx(idxd_v, blk)], xr_vs[b], gsems[b])
        pltpu.async_copy(xe_h.at[pl.ds(base, B)], xe_vs[b], gsems[b])

    def wait_gathers(b, blk):
        base = pl.multiple_of(wid * EPT, 8) + blk * B
        pltpu.make_async_copy(xl_h.at[_idx(idxs_v, blk)], xl_vs[b], gsems[b]).wait()
        pltpu.make_async_copy(xr_h.at[_idx(idxd_v, blk)], xr_vs[b], gsems[b]).wait()
        pltpu.make_async_copy(xe_h.at[pl.ds(base, B)], xe_vs[b], gsems[b]).wait()

    def compute_block(b):
        xl_v, xr_v, xe_v, out_v = xl_vs[b], xr_vs[b], xe_vs[b], out_vs[b]

        def ebody(e, ecarry):
            xlc = []
            acc = z
            for c4 in range(4):
                vl = xl_v[e, c4]
                vr = xr_v[e, c4]
                ve = xe_v[e, c4]
                for h, unpack in enumerate((_lo, _hi)):
                    xlh = unpack(vl)
                    xlc.append(xlh)
                    m = xlh + unpack(vr) + unpack(ve)
                    m = jnp.maximum(m, 0.2 * m)
                    acc = acc + m * attc[2 * c4 + h]
            ex = jnp.exp(_lanesum(acc))
            out_v[e, pl.ds(ROW - 16, 16)] = ex
            for c in range(8):
                out_v[e, pl.ds(c * 16, 16)] = xlc[c] * ex
            return ecarry

        lax.fori_loop(0, B, ebody, 0)

    issue_gathers(0, 0)
    issue_gathers(1, 1)

    def grp_body(g, carry):
        for b in range(2):
            blk = 2 * g + b
            wait_gathers(b, blk)

            @pl.when(blk >= 2)
            def _():
                pltpu.make_async_copy(
                    out_vs[b], acc_sh.at[_idx(idxd_v, blk - 2)], ssems[b]
                ).wait()

            compute_block(b)
            pltpu.async_copy(out_vs[b], acc_sh.at[_idx(idxd_v, blk)], ssems[b],
                             add=True)

            @pl.when(blk + 2 < NBLK)
            def _():
                @pl.when((blk + 2) % IC == 0)
                def _():
                    wait_idx_chunk((blk + 2) // IC)

                issue_gathers(b, blk + 2)

            if b == 0:
                @pl.when(jnp.logical_and(blk % IC == 0, blk + IC < NBLK))
                def _():
                    issue_idx_chunk(blk // IC + 1)

        return carry

    lax.fori_loop(0, NBLK // 2, grp_body, 0)

    pltpu.make_async_copy(
        out_v0, acc_sh.at[_idx(idxd_v, NBLK - 2)], ssem0).wait()
    pltpu.make_async_copy(
        out_v1, acc_sh.at[_idx(idxd_v, NBLK - 1)], ssem1).wait()
    plsc.subcore_barrier()

    def exp_body(k, carry):
        j = sid + k * NS

        @pl.when(j < NROWCHUNK)
        def _():
            pltpu.sync_copy(acc_sh.at[pl.ds(j * B, B)], out_v0)
            pltpu.sync_copy(out_v0, acc_h.at[cid, pl.ds(j * B, B)])

        return carry

    lax.fori_loop(0, (NROWCHUNK + NS - 1) // NS, exp_body, 0)


_edge_pass = pl.kernel(
    _edge_pass_body,
    out_type=jax.ShapeDtypeStruct((NC, N, ROW), jnp.float32),
    mesh=plsc.VectorSubcoreMesh(core_axis_name="c", subcore_axis_name="s"),
    compiler_params=pltpu.CompilerParams(
        needs_layout_passes=False, use_tc_tiling_on_sc=False
    ),
    scratch_types=[
        pltpu.VMEM((2, IC, B), jnp.int32),
        pltpu.VMEM((2, IC, B), jnp.int32),
        pltpu.VMEM((B, 4, 16), jnp.int32),
        pltpu.VMEM((B, 4, 16), jnp.int32),
        pltpu.VMEM((B, 4, 16), jnp.int32),
        pltpu.VMEM((B, 4, 16), jnp.int32),
        pltpu.VMEM((B, 4, 16), jnp.int32),
        pltpu.VMEM((B, 4, 16), jnp.int32),
        pltpu.VMEM((B, ROW), jnp.float32),
        pltpu.VMEM((B, ROW), jnp.float32),
        pltpu.VMEM((D,), jnp.float32),
        pltpu.VMEM_SHARED((N, ROW), jnp.float32),
        pltpu.SemaphoreType.DMA,
        pltpu.SemaphoreType.DMA,
        pltpu.SemaphoreType.DMA,
        pltpu.SemaphoreType.DMA,
        pltpu.SemaphoreType.DMA,
    ],
)


_PERM = np.empty(D, np.int32)
for _g in range(4):
    for _i in range(16):
        _PERM[32 * _g + 2 * _i] = 32 * _g + _i
        _PERM[32 * _g + 2 * _i + 1] = 32 * _g + 16 + _i


def _as_i32(a, n):
    return lax.bitcast_convert_type(
        a.reshape(n, D // 2, 2), jnp.int32).reshape(n, D // 32, 16)


def kernel(x, edge_index, edge_attr, Wl1, Wr1, We1, att1, b1, Wl2, Wr2, We2, att2, b2):
    src = edge_index[0].reshape(NW, NBLK, B)
    dst = edge_index[1].reshape(NW, NBLK, B)
    xl1, xr1 = _node_mm(x, Wl1[:, _PERM], Wr1[:, _PERM])
    xe1 = _edge_mm(edge_attr, We1[:, _PERM])
    acc1 = _edge_pass(src, dst, _as_i32(xl1, N), _as_i32(xr1, N),
                      _as_i32(xe1, E), att1.reshape(D))
    xe2 = _edge_mm(edge_attr, We2[:, _PERM])
    xl2, xr2 = _combine_mm(acc1, b1.reshape(1, D), Wl2[:, _PERM], Wr2[:, _PERM])
    acc2 = _edge_pass(src, dst, _as_i32(xl2, N), _as_i32(xr2, N),
                      _as_i32(xe2, E), att2.reshape(D))
    return _combine_final(acc2, b2.reshape(1, D))


# speedup vs baseline: 1.1808x; 1.1808x over previous
import functools

import jax
import jax.numpy as jnp
import numpy as np
from jax import lax
from jax.experimental import pallas as pl
from jax.experimental.pallas import tpu as pltpu
from jax.experimental.pallas import tpu_sc as plsc

N = 10000
E = 320000
D = 128
ROW = 136
NC = 2
NS = 16
NW = NC * NS
EPT = E // NW
B = 40
NBLK = EPT // B
IC = 10
NCHUNK = NBLK // IC
NROWCHUNK = (N + B - 1) // B


def _node_mm_body(x_ref, wl_ref, wr_ref, xl_ref, xr_ref):
    xb = x_ref[...]
    xl_ref[...] = jnp.dot(
        xb, wl_ref[...], preferred_element_type=jnp.float32
    ).astype(jnp.bfloat16)
    xr_ref[...] = jnp.dot(
        xb, wr_ref[...], preferred_element_type=jnp.float32
    ).astype(jnp.bfloat16)


_node_mm = pl.pallas_call(
    _node_mm_body,
    grid=(10,),
    in_specs=[
        pl.BlockSpec((1000, D), lambda i: (i, 0)),
        pl.BlockSpec((D, D), lambda i: (0, 0)),
        pl.BlockSpec((D, D), lambda i: (0, 0)),
    ],
    out_specs=[pl.BlockSpec((1000, D), lambda i: (i, 0))] * 2,
    out_shape=[jax.ShapeDtypeStruct((N, D), jnp.bfloat16)] * 2,
)


def _edge_mm_body(ea_ref, we_ref, xe_ref):
    xe_ref[...] = jnp.dot(ea_ref[...], we_ref[...],
                          preferred_element_type=jnp.float32).astype(jnp.bfloat16)


_edge_mm = pl.pallas_call(
    _edge_mm_body,
    grid=(80,),
    in_specs=[
        pl.BlockSpec((4000, 16), lambda i: (i, 0)),
        pl.BlockSpec((16, D), lambda i: (0, 0)),
    ],
    out_specs=pl.BlockSpec((4000, D), lambda i: (i, 0)),
    out_shape=jax.ShapeDtypeStruct((E, D), jnp.bfloat16),
)


def _combine_mm_body(acc_ref, b_ref, wl_ref, wr_ref, xl_ref, xr_ref):
    s = acc_ref[0] + acc_ref[1]
    num = s[:, :D]
    den = s[:, D:D + 1]
    h = num / (den + 1e-16) + b_ref[...]
    h = jnp.where(h > 0, h, jnp.exp(h) - 1.0)
    xl_ref[...] = jnp.dot(
        h, wl_ref[...], preferred_element_type=jnp.float32
    ).astype(jnp.bfloat16)
    xr_ref[...] = jnp.dot(
        h, wr_ref[...], preferred_element_type=jnp.float32
    ).astype(jnp.bfloat16)


_combine_mm = pl.pallas_call(
    _combine_mm_body,
    grid=(10,),
    in_specs=[
        pl.BlockSpec((2, 1000, ROW), lambda i: (0, i, 0)),
        pl.BlockSpec((1, D), lambda i: (0, 0)),
        pl.BlockSpec((D, D), lambda i: (0, 0)),
        pl.BlockSpec((D, D), lambda i: (0, 0)),
    ],
    out_specs=[pl.BlockSpec((1000, D), lambda i: (i, 0))] * 2,
    out_shape=[jax.ShapeDtypeStruct((N, D), jnp.bfloat16)] * 2,
)


def _combine_final_body(acc_ref, b_ref, out_ref):
    s = acc_ref[0] + acc_ref[1]
    num = s[:, :D]
    den = s[:, D:D + 1]
    out_ref[...] = num / (den + 1e-16) + b_ref[...]


_combine_final = pl.pallas_call(
    _combine_final_body,
    grid=(10,),
    in_specs=[
        pl.BlockSpec((2, 1000, ROW), lambda i: (0, i, 0)),
        pl.BlockSpec((1, D), lambda i: (0, 0)),
    ],
    out_specs=pl.BlockSpec((1000, D), lambda i: (i, 0)),
    out_shape=jax.ShapeDtypeStruct((N, D), jnp.float32),
)


def _edge_pass_body(src_h, dst_h, xl_h, xr_h, xe_h, att_h, acc_h,
                    idxs_v, idxd_v, xl_v0, xl_v1, xr_v0, xr_v1, xe_v0, xe_v1,
                    out_v0, out_v1, att_v, acc_sh,
                    gsem0, gsem1, ssem0, ssem1, isem):
    cid = lax.axis_index("c")
    sid = lax.axis_index("s")
    wid = cid * NS + sid
    xl_vs = (xl_v0, xl_v1)
    xr_vs = (xr_v0, xr_v1)
    xe_vs = (xe_v0, xe_v1)
    out_vs = (out_v0, out_v1)
    gsems = (gsem0, gsem1)
    ssems = (ssem0, ssem1)

    z = jnp.zeros((16,), jnp.float32)

    def zv_body(i, carry):
        out_v0[i, pl.ds(ROW - 16, 16)] = z
        for c in range(8):
            out_v0[i, pl.ds(c * 16, 16)] = z
        return carry

    lax.fori_loop(0, B, zv_body, 0)

    def zacc_body(k, carry):
        j = sid + k * NS

        @pl.when(j < NROWCHUNK)
        def _():
            pltpu.sync_copy(out_v0, acc_sh.at[pl.ds(j * B, B)])

        return carry

    lax.fori_loop(0, (NROWCHUNK + NS - 1) // NS, zacc_body, 0)

    pltpu.sync_copy(src_h.at[wid, pl.ds(0, IC)], idxs_v.at[0])
    pltpu.sync_copy(dst_h.at[wid, pl.ds(0, IC)], idxd_v.at[0])
    pltpu.sync_copy(att_h, att_v)
    plsc.subcore_barrier()

    attc = [att_v[pl.ds(c * 16, 16)] for c in range(8)]
    shift16 = jnp.full((16,), 16, jnp.int32)
    himask = jnp.full((16,), -0x10000, jnp.int32)

    def _lo(v):
        return lax.bitcast_convert_type(lax.shift_left(v, shift16), jnp.float32)

    def _hi(v):
        return lax.bitcast_convert_type(jnp.bitwise_and(v, himask), jnp.float32)

    def _lanesum(v):
        return lax.broadcast(jnp.sum(v), (16,))

    def _idx(ref, blk):
        return ref.at[(blk // IC) % 2, blk % IC]

    def issue_idx_chunk(c):
        pltpu.async_copy(src_h.at[wid, pl.ds(c * IC, IC)], idxs_v.at[c % 2], isem)
        pltpu.async_copy(dst_h.at[wid, pl.ds(c * IC, IC)], idxd_v.at[c % 2], isem)

    def wait_idx_chunk(c):
        pltpu.make_async_copy(
            src_h.at[wid, pl.ds(c * IC, IC)], idxs_v.at[c % 2], isem).wait()
        pltpu.make_async_copy(
            dst_h.at[wid, pl.ds(c * IC, IC)], idxd_v.at[c % 2], isem).wait()

    def issue_gathers(b, blk):
        base = pl.multiple_of(wid * EPT, 8) + blk * B
        pltpu.async_copy(xl_h.at[_idx(idxs_v, blk)], xl_vs[b], gsems[b])
        pltpu.async_copy(xr_h.at[_idx(idxd_v, blk)], xr_vs[b], gsems[b])
        pltpu.async_copy(xe_h.at[pl.ds(base, B)], xe_vs[b], gsems[b])

    def wait_gathers(b, blk):
        base = pl.multiple_of(wid * EPT, 8) + blk * B
        pltpu.make_async_copy(xl_h.at[_idx(idxs_v, blk)], xl_vs[b], gsems[b]).wait()
        pltpu.make_async_copy(xr_h.at[_idx(idxd_v, blk)], xr_vs[b], gsems[b]).wait()
        pltpu.make_async_copy(xe_h.at[pl.ds(base, B)], xe_vs[b], gsems[b]).wait()

    def compute_block(b):
        xl_v, xr_v, xe_v, out_v = xl_vs[b], xr_vs[b], xe_vs[b], out_vs[b]

        def ebody(e, ecarry):
            xlc = []
            acc = z
            for c4 in range(4):
                vl = xl_v[e, pl.ds(c4 * 16, 16)]
                vr = xr_v[e, pl.ds(c4 * 16, 16)]
                ve = xe_v[e, pl.ds(c4 * 16, 16)]
                for h, unpack in enumerate((_lo, _hi)):
                    xlh = unpack(vl)
                    xlc.append(xlh)
                    m = xlh + unpack(vr) + unpack(ve)
                    m = jnp.maximum(m, 0.2 * m)
                    acc = acc + m * attc[2 * c4 + h]
            ex = jnp.exp(_lanesum(acc))
            out_v[e, pl.ds(ROW - 16, 16)] = ex
            for c in range(8):
                out_v[e, pl.ds(c * 16, 16)] = xlc[c] * ex
            return ecarry

        lax.fori_loop(0, B, ebody, 0)

    issue_gathers(0, 0)
    issue_gathers(1, 1)

    def grp_body(g, carry):
        for b in range(2):
            blk = 2 * g + b
            wait_gathers(b, blk)

            @pl.when(blk >= 2)
            def _():
                pltpu.make_async_copy(
                    out_vs[b], acc_sh.at[_idx(idxd_v, blk - 2)], ssems[b]
                ).wait()

            compute_block(b)
            pltpu.async_copy(out_vs[b], acc_sh.at[_idx(idxd_v, blk)], ssems[b],
                             add=True)

            @pl.when(blk + 2 < NBLK)
            def _():
                @pl.when((blk + 2) % IC == 0)
                def _():
                    wait_idx_chunk((blk + 2) // IC)

                issue_gathers(b, blk + 2)

            if b == 0:
                @pl.when(jnp.logical_and(blk % IC == 0, blk + IC < NBLK))
                def _():
                    issue_idx_chunk(blk // IC + 1)

        return carry

    lax.fori_loop(0, NBLK // 2, grp_body, 0)

    pltpu.make_async_copy(
        out_v0, acc_sh.at[_idx(idxd_v, NBLK - 2)], ssem0).wait()
    pltpu.make_async_copy(
        out_v1, acc_sh.at[_idx(idxd_v, NBLK - 1)], ssem1).wait()
    plsc.subcore_barrier()

    def exp_body(k, carry):
        j = sid + k * NS

        @pl.when(j < NROWCHUNK)
        def _():
            pltpu.sync_copy(acc_sh.at[pl.ds(j * B, B)], out_v0)
            pltpu.sync_copy(out_v0, acc_h.at[cid, pl.ds(j * B, B)])

        return carry

    lax.fori_loop(0, (NROWCHUNK + NS - 1) // NS, exp_body, 0)


_edge_pass = pl.kernel(
    _edge_pass_body,
    out_type=jax.ShapeDtypeStruct((NC, N, ROW), jnp.float32),
    mesh=plsc.VectorSubcoreMesh(core_axis_name="c", subcore_axis_name="s"),
    compiler_params=pltpu.CompilerParams(
        needs_layout_passes=False, use_tc_tiling_on_sc=False
    ),
    scratch_types=[
        pltpu.VMEM((2, IC, B), jnp.int32),
        pltpu.VMEM((2, IC, B), jnp.int32),
        pltpu.VMEM((B, 64), jnp.int32),
        pltpu.VMEM((B, 64), jnp.int32),
        pltpu.VMEM((B, 64), jnp.int32),
        pltpu.VMEM((B, 64), jnp.int32),
        pltpu.VMEM((B, 64), jnp.int32),
        pltpu.VMEM((B, 64), jnp.int32),
        pltpu.VMEM((B, ROW), jnp.float32),
        pltpu.VMEM((B, ROW), jnp.float32),
        pltpu.VMEM((D,), jnp.float32),
        pltpu.VMEM_SHARED((N, ROW), jnp.float32),
        pltpu.SemaphoreType.DMA,
        pltpu.SemaphoreType.DMA,
        pltpu.SemaphoreType.DMA,
        pltpu.SemaphoreType.DMA,
        pltpu.SemaphoreType.DMA,
    ],
)


_PERM = np.empty(D, np.int32)
for _g in range(4):
    for _i in range(16):
        _PERM[32 * _g + 2 * _i] = 32 * _g + _i
        _PERM[32 * _g + 2 * _i + 1] = 32 * _g + 16 + _i


def _as_i32(a, n):
    return lax.bitcast_convert_type(a.reshape(n, D // 2, 2), jnp.int32)


def kernel(x, edge_index, edge_attr, Wl1, Wr1, We1, att1, b1, Wl2, Wr2, We2, att2, b2):
    src = edge_index[0].reshape(NW, NBLK, B)
    dst = edge_index[1].reshape(NW, NBLK, B)
    xl1, xr1 = _node_mm(x, Wl1[:, _PERM], Wr1[:, _PERM])
    xe1 = _edge_mm(edge_attr, We1[:, _PERM])
    acc1 = _edge_pass(src, dst, _as_i32(xl1, N), _as_i32(xr1, N),
                      _as_i32(xe1, E), att1.reshape(D))
    xe2 = _edge_mm(edge_attr, We2[:, _PERM])
    xl2, xr2 = _combine_mm(acc1, b1.reshape(1, D), Wl2[:, _PERM], Wr2[:, _PERM])
    acc2 = _edge_pass(src, dst, _as_i32(xl2, N), _as_i32(xr2, N),
                      _as_i32(xe2, E), att2.reshape(D))
    return _combine_final(acc2, b2.reshape(1, D))


# speedup vs baseline: 2.9616x; 2.5082x over previous
import functools

import jax
import jax.numpy as jnp
from jax import lax
from jax.experimental import pallas as pl
from jax.experimental.pallas import tpu as pltpu
from jax.experimental.pallas import tpu_sc as plsc

N = 10000
E = 320000
D = 128
ROW = 136
NC = 2
NS = 16
NW = NC * NS
EPT = E // NW
B = 40
NBLK = EPT // B
IC = 10
NCHUNK = NBLK // IC
NROWCHUNK = (N + B - 1) // B


def _node_mm_body(x_ref, wl_ref, wr_ref, xl_ref, xr_ref):
    xb = x_ref[...]
    xl_ref[...] = jnp.dot(xb, wl_ref[...], preferred_element_type=jnp.float32)
    xr_ref[...] = jnp.dot(xb, wr_ref[...], preferred_element_type=jnp.float32)


_node_mm = pl.pallas_call(
    _node_mm_body,
    grid=(10,),
    in_specs=[
        pl.BlockSpec((1000, D), lambda i: (i, 0)),
        pl.BlockSpec((D, D), lambda i: (0, 0)),
        pl.BlockSpec((D, D), lambda i: (0, 0)),
    ],
    out_specs=[pl.BlockSpec((1000, D), lambda i: (i, 0))] * 2,
    out_shape=[jax.ShapeDtypeStruct((N, D), jnp.float32)] * 2,
)


def _edge_mm_body(ea_ref, we_ref, xe_ref):
    xe_ref[...] = jnp.dot(ea_ref[...], we_ref[...],
                          preferred_element_type=jnp.float32)


_edge_mm = pl.pallas_call(
    _edge_mm_body,
    grid=(80,),
    in_specs=[
        pl.BlockSpec((4000, 16), lambda i: (i, 0)),
        pl.BlockSpec((16, D), lambda i: (0, 0)),
    ],
    out_specs=pl.BlockSpec((4000, D), lambda i: (i, 0)),
    out_shape=jax.ShapeDtypeStruct((E, D), jnp.float32),
)


def _combine_mm_body(acc_ref, b_ref, wl_ref, wr_ref, xl_ref, xr_ref):
    s = acc_ref[0] + acc_ref[1]
    num = s[:, :D]
    den = s[:, D:D + 1]
    h = num / (den + 1e-16) + b_ref[...]
    h = jnp.where(h > 0, h, jnp.exp(h) - 1.0)
    xl_ref[...] = jnp.dot(h, wl_ref[...], preferred_element_type=jnp.float32)
    xr_ref[...] = jnp.dot(h, wr_ref[...], preferred_element_type=jnp.float32)


_combine_mm = pl.pallas_call(
    _combine_mm_body,
    grid=(10,),
    in_specs=[
        pl.BlockSpec((2, 1000, ROW), lambda i: (0, i, 0)),
        pl.BlockSpec((1, D), lambda i: (0, 0)),
        pl.BlockSpec((D, D), lambda i: (0, 0)),
        pl.BlockSpec((D, D), lambda i: (0, 0)),
    ],
    out_specs=[pl.BlockSpec((1000, D), lambda i: (i, 0))] * 2,
    out_shape=[jax.ShapeDtypeStruct((N, D), jnp.float32)] * 2,
)


def _combine_final_body(acc_ref, b_ref, out_ref):
    s = acc_ref[0] + acc_ref[1]
    num = s[:, :D]
    den = s[:, D:D + 1]
    out_ref[...] = num / (den + 1e-16) + b_ref[...]


_combine_final = pl.pallas_call(
    _combine_final_body,
    grid=(10,),
    in_specs=[
        pl.BlockSpec((2, 1000, ROW), lambda i: (0, i, 0)),
        pl.BlockSpec((1, D), lambda i: (0, 0)),
    ],
    out_specs=pl.BlockSpec((1000, D), lambda i: (i, 0)),
    out_shape=jax.ShapeDtypeStruct((N, D), jnp.float32),
)


def _edge_pass_body(src_h, dst_h, xl_h, xr_h, xe_h, att_h, acc_h,
                    idxs_v, idxd_v, xl_v0, xl_v1, xr_v0, xr_v1, xe_v0, xe_v1,
                    out_v0, out_v1, att_v, acc_sh,
                    gsem0, gsem1, ssem0, ssem1, isem):
    cid = lax.axis_index("c")
    sid = lax.axis_index("s")
    wid = cid * NS + sid
    xl_vs = (xl_v0, xl_v1)
    xr_vs = (xr_v0, xr_v1)
    xe_vs = (xe_v0, xe_v1)
    out_vs = (out_v0, out_v1)
    gsems = (gsem0, gsem1)
    ssems = (ssem0, ssem1)

    z = jnp.zeros((16,), jnp.float32)

    def zv_body(i, carry):
        out_v0[i, pl.ds(ROW - 16, 16)] = z
        for c in range(8):
            out_v0[i, pl.ds(c * 16, 16)] = z
        return carry

    lax.fori_loop(0, B, zv_body, 0)

    def zacc_body(k, carry):
        j = sid + k * NS

        @pl.when(j < NROWCHUNK)
        def _():
            pltpu.sync_copy(out_v0, acc_sh.at[pl.ds(j * B, B)])

        return carry

    lax.fori_loop(0, (NROWCHUNK + NS - 1) // NS, zacc_body, 0)

    pltpu.sync_copy(src_h.at[wid, pl.ds(0, IC)], idxs_v.at[0])
    pltpu.sync_copy(dst_h.at[wid, pl.ds(0, IC)], idxd_v.at[0])
    pltpu.sync_copy(att_h, att_v)
    plsc.subcore_barrier()

    attc = [att_v[pl.ds(c * 16, 16)] for c in range(8)]

    def _lanesum(v):
        return lax.broadcast(jnp.sum(v), (16,))

    def _idx(ref, blk):
        return ref.at[(blk // IC) % 2, blk % IC]

    def issue_idx_chunk(c):
        pltpu.async_copy(src_h.at[wid, pl.ds(c * IC, IC)], idxs_v.at[c % 2], isem)
        pltpu.async_copy(dst_h.at[wid, pl.ds(c * IC, IC)], idxd_v.at[c % 2], isem)

    def wait_idx_chunk(c):
        pltpu.make_async_copy(
            src_h.at[wid, pl.ds(c * IC, IC)], idxs_v.at[c % 2], isem).wait()
        pltpu.make_async_copy(
            dst_h.at[wid, pl.ds(c * IC, IC)], idxd_v.at[c % 2], isem).wait()

    def issue_gathers(b, blk):
        base = pl.multiple_of(wid * EPT, 8) + blk * B
        pltpu.async_copy(xl_h.at[_idx(idxs_v, blk)], xl_vs[b], gsems[b])
        pltpu.async_copy(xr_h.at[_idx(idxd_v, blk)], xr_vs[b], gsems[b])
        pltpu.async_copy(xe_h.at[pl.ds(base, B)], xe_vs[b], gsems[b])

    def wait_gathers(b, blk):
        base = pl.multiple_of(wid * EPT, 8) + blk * B
        pltpu.make_async_copy(xl_h.at[_idx(idxs_v, blk)], xl_vs[b], gsems[b]).wait()
        pltpu.make_async_copy(xr_h.at[_idx(idxd_v, blk)], xr_vs[b], gsems[b]).wait()
        pltpu.make_async_copy(xe_h.at[pl.ds(base, B)], xe_vs[b], gsems[b]).wait()

    def compute_block(b):
        xl_v, xr_v, xe_v, out_v = xl_vs[b], xr_vs[b], xe_vs[b], out_vs[b]

        def ebody(i, ecarry):
            for e in (2 * i, 2 * i + 1):
                xlc = [xl_v[e, pl.ds(c * 16, 16)] for c in range(8)]
                acc = z
                for c in range(8):
                    m = xlc[c] + xr_v[e, pl.ds(c * 16, 16)] + xe_v[e, pl.ds(c * 16, 16)]
                    m = jnp.maximum(m, 0.2 * m)
                    acc = acc + m * attc[c]
                ex = jnp.exp(_lanesum(acc))
                out_v[e, pl.ds(ROW - 16, 16)] = ex
                for c in range(8):
                    out_v[e, pl.ds(c * 16, 16)] = xlc[c] * ex
            return ecarry

        lax.fori_loop(0, B // 2, ebody, 0)

    issue_gathers(0, 0)
    issue_gathers(1, 1)

    def grp_body(g, carry):
        for b in range(2):
            blk = 2 * g + b
            wait_gathers(b, blk)

            @pl.when(blk >= 2)
            def _():
                pltpu.make_async_copy(
                    out_vs[b], acc_sh.at[_idx(idxd_v, blk - 2)], ssems[b]
                ).wait()

            compute_block(b)
            pltpu.async_copy(out_vs[b], acc_sh.at[_idx(idxd_v, blk)], ssems[b],
                             add=True)

            @pl.when(blk + 2 < NBLK)
            def _():
                @pl.when((blk + 2) % IC == 0)
                def _():
                    wait_idx_chunk((blk + 2) // IC)

                issue_gathers(b, blk + 2)

            if b == 0:
                @pl.when(jnp.logical_and(blk % IC == 0, blk + IC < NBLK))
                def _():
                    issue_idx_chunk(blk // IC + 1)

        return carry

    lax.fori_loop(0, NBLK // 2, grp_body, 0)

    pltpu.make_async_copy(
        out_v0, acc_sh.at[_idx(idxd_v, NBLK - 2)], ssem0).wait()
    pltpu.make_async_copy(
        out_v1, acc_sh.at[_idx(idxd_v, NBLK - 1)], ssem1).wait()
    plsc.subcore_barrier()

    def exp_body(k, carry):
        j = sid + k * NS

        @pl.when(j < NROWCHUNK)
        def _():
            pltpu.sync_copy(acc_sh.at[pl.ds(j * B, B)], out_v0)
            pltpu.sync_copy(out_v0, acc_h.at[cid, pl.ds(j * B, B)])

        return carry

    lax.fori_loop(0, (NROWCHUNK + NS - 1) // NS, exp_body, 0)


_edge_pass = pl.kernel(
    _edge_pass_body,
    out_type=jax.ShapeDtypeStruct((NC, N, ROW), jnp.float32),
    mesh=plsc.VectorSubcoreMesh(core_axis_name="c", subcore_axis_name="s"),
    compiler_params=pltpu.CompilerParams(
        needs_layout_passes=False, use_tc_tiling_on_sc=False
    ),
    scratch_types=[
        pltpu.VMEM((2, IC, B), jnp.int32),
        pltpu.VMEM((2, IC, B), jnp.int32),
        pltpu.VMEM((B, D), jnp.float32),
        pltpu.VMEM((B, D), jnp.float32),
        pltpu.VMEM((B, D), jnp.float32),
        pltpu.VMEM((B, D), jnp.float32),
        pltpu.VMEM((B, D), jnp.float32),
        pltpu.VMEM((B, D), jnp.float32),
        pltpu.VMEM((B, ROW), jnp.float32),
        pltpu.VMEM((B, ROW), jnp.float32),
        pltpu.VMEM((D,), jnp.float32),
        pltpu.VMEM_SHARED((N, ROW), jnp.float32),
        pltpu.SemaphoreType.DMA,
        pltpu.SemaphoreType.DMA,
        pltpu.SemaphoreType.DMA,
        pltpu.SemaphoreType.DMA,
        pltpu.SemaphoreType.DMA,
    ],
)


def kernel(x, edge_index, edge_attr, Wl1, Wr1, We1, att1, b1, Wl2, Wr2, We2, att2, b2):
    src = edge_index[0].reshape(NW, NBLK, B)
    dst = edge_index[1].reshape(NW, NBLK, B)
    xl1, xr1 = _node_mm(x, Wl1, Wr1)
    xe1 = _edge_mm(edge_attr, We1)
    acc1 = _edge_pass(src, dst, xl1, xr1, xe1, att1.reshape(D))
    xe2 = _edge_mm(edge_attr, We2)
    xl2, xr2 = _combine_mm(acc1, b1.reshape(1, D), Wl2, Wr2)
    acc2 = _edge_pass(src, dst, xl2, xr2, xe2, att2.reshape(D))
    return _combine_final(acc2, b2.reshape(1, D))


# speedup vs baseline: 3.1336x; 1.0581x over previous
import functools

import jax
import jax.numpy as jnp
from jax import lax
from jax.experimental import pallas as pl
from jax.experimental.pallas import tpu as pltpu
from jax.experimental.pallas import tpu_sc as plsc

N = 10000
E = 320000
D = 128
ROW = 136
NC = 2
NS = 16
NW = NC * NS
EPT = E // NW
B = 40
NBLK = EPT // B
IC = 10
NCHUNK = NBLK // IC
NROWCHUNK = (N + B - 1) // B


def _node_mm_body(x_ref, wl_ref, wr_ref, xl_ref, xr_ref):
    xb = x_ref[...]
    xl_ref[...] = jnp.dot(xb, wl_ref[...], preferred_element_type=jnp.float32)
    xr_ref[...] = jnp.dot(xb, wr_ref[...], preferred_element_type=jnp.float32)


_node_mm = pl.pallas_call(
    _node_mm_body,
    grid=(10,),
    in_specs=[
        pl.BlockSpec((1000, D), lambda i: (i, 0)),
        pl.BlockSpec((D, D), lambda i: (0, 0)),
        pl.BlockSpec((D, D), lambda i: (0, 0)),
    ],
    out_specs=[pl.BlockSpec((1000, D), lambda i: (i, 0))] * 2,
    out_shape=[jax.ShapeDtypeStruct((N, D), jnp.float32)] * 2,
)


def _edge_mm_body(ea_ref, we_ref, xe_ref):
    a = jnp.dot(ea_ref[...], we_ref[...], preferred_element_type=jnp.float32)
    xe_ref[...] = pltpu.bitcast(a.astype(jnp.bfloat16), jnp.int32)


_edge_mm = pl.pallas_call(
    _edge_mm_body,
    grid=(80,),
    in_specs=[
        pl.BlockSpec((4000, 16), lambda i: (i, 0)),
        pl.BlockSpec((16, D), lambda i: (0, 0)),
    ],
    out_specs=pl.BlockSpec((2000, D), lambda i: (i, 0)),
    out_shape=jax.ShapeDtypeStruct((E // 2, D), jnp.int32),
)


def _combine_mm_body(acc_ref, b_ref, wl_ref, wr_ref, xl_ref, xr_ref):
    s = acc_ref[0] + acc_ref[1]
    num = s[:, :D]
    den = s[:, D:D + 1]
    h = num / (den + 1e-16) + b_ref[...]
    h = jnp.where(h > 0, h, jnp.exp(h) - 1.0)
    xl_ref[...] = jnp.dot(h, wl_ref[...], preferred_element_type=jnp.float32)
    xr_ref[...] = jnp.dot(h, wr_ref[...], preferred_element_type=jnp.float32)


_combine_mm = pl.pallas_call(
    _combine_mm_body,
    grid=(10,),
    in_specs=[
        pl.BlockSpec((2, 1000, ROW), lambda i: (0, i, 0)),
        pl.BlockSpec((1, D), lambda i: (0, 0)),
        pl.BlockSpec((D, D), lambda i: (0, 0)),
        pl.BlockSpec((D, D), lambda i: (0, 0)),
    ],
    out_specs=[pl.BlockSpec((1000, D), lambda i: (i, 0))] * 2,
    out_shape=[jax.ShapeDtypeStruct((N, D), jnp.float32)] * 2,
)


def _combine_final_body(acc_ref, b_ref, out_ref):
    s = acc_ref[0] + acc_ref[1]
    num = s[:, :D]
    den = s[:, D:D + 1]
    out_ref[...] = num / (den + 1e-16) + b_ref[...]


_combine_final = pl.pallas_call(
    _combine_final_body,
    grid=(10,),
    in_specs=[
        pl.BlockSpec((2, 1000, ROW), lambda i: (0, i, 0)),
        pl.BlockSpec((1, D), lambda i: (0, 0)),
    ],
    out_specs=pl.BlockSpec((1000, D), lambda i: (i, 0)),
    out_shape=jax.ShapeDtypeStruct((N, D), jnp.float32),
)


def _edge_pass_body(src_h, dst_h, xl_h, xr_h, xe_h, att_h, acc_h,
                    idxs_v, idxd_v, xl_v0, xl_v1, xr_v0, xr_v1, xe_v0, xe_v1,
                    out_v0, out_v1, att_v, acc_sh,
                    gsem0, gsem1, ssem0, ssem1, isem):
    cid = lax.axis_index("c")
    sid = lax.axis_index("s")
    wid = cid * NS + sid
    xl_vs = (xl_v0, xl_v1)
    xr_vs = (xr_v0, xr_v1)
    xe_vs = (xe_v0, xe_v1)
    out_vs = (out_v0, out_v1)
    gsems = (gsem0, gsem1)
    ssems = (ssem0, ssem1)

    z = jnp.zeros((16,), jnp.float32)

    def zv_body(i, carry):
        out_v0[i, pl.ds(ROW - 16, 16)] = z
        for c in range(8):
            out_v0[i, pl.ds(c * 16, 16)] = z
        return carry

    lax.fori_loop(0, B, zv_body, 0)

    def zacc_body(k, carry):
        j = sid + k * NS

        @pl.when(j < NROWCHUNK)
        def _():
            pltpu.sync_copy(out_v0, acc_sh.at[pl.ds(j * B, B)])

        return carry

    lax.fori_loop(0, (NROWCHUNK + NS - 1) // NS, zacc_body, 0)

    pltpu.sync_copy(src_h.at[wid, pl.ds(0, IC)], idxs_v.at[0])
    pltpu.sync_copy(dst_h.at[wid, pl.ds(0, IC)], idxd_v.at[0])
    pltpu.sync_copy(att_h, att_v)
    plsc.subcore_barrier()

    attc = [att_v[pl.ds(c * 16, 16)] for c in range(8)]

    def _lanesum(v):
        return lax.broadcast(jnp.sum(v), (16,))

    def _idx(ref, blk):
        return ref.at[(blk // IC) % 2, blk % IC]

    def issue_idx_chunk(c):
        pltpu.async_copy(src_h.at[wid, pl.ds(c * IC, IC)], idxs_v.at[c % 2], isem)
        pltpu.async_copy(dst_h.at[wid, pl.ds(c * IC, IC)], idxd_v.at[c % 2], isem)

    def wait_idx_chunk(c):
        pltpu.make_async_copy(
            src_h.at[wid, pl.ds(c * IC, IC)], idxs_v.at[c % 2], isem).wait()
        pltpu.make_async_copy(
            dst_h.at[wid, pl.ds(c * IC, IC)], idxd_v.at[c % 2], isem).wait()

    def issue_gathers(b, blk):
        base = pl.multiple_of(wid * (EPT // 2), 4) + blk * (B // 2)
        pltpu.async_copy(xl_h.at[_idx(idxs_v, blk)], xl_vs[b], gsems[b])
        pltpu.async_copy(xr_h.at[_idx(idxd_v, blk)], xr_vs[b], gsems[b])
        pltpu.async_copy(xe_h.at[pl.ds(base, B // 2)], xe_vs[b], gsems[b])

    def wait_gathers(b, blk):
        base = pl.multiple_of(wid * (EPT // 2), 4) + blk * (B // 2)
        pltpu.make_async_copy(xl_h.at[_idx(idxs_v, blk)], xl_vs[b], gsems[b]).wait()
        pltpu.make_async_copy(xr_h.at[_idx(idxd_v, blk)], xr_vs[b], gsems[b]).wait()
        pltpu.make_async_copy(xe_h.at[pl.ds(base, B // 2)], xe_vs[b], gsems[b]).wait()

    def compute_block(b):
        xl_v, xr_v, xe_v, out_v = xl_vs[b], xr_vs[b], xe_vs[b], out_vs[b]

        def ebody(i, ecarry):
            xew = [xe_v[i, pl.ds(c * 16, 16)] for c in range(8)]
            xe_lo = [lax.bitcast_convert_type(w << 16, jnp.float32) for w in xew]
            xe_hi = [lax.bitcast_convert_type(w & jnp.int32(-65536), jnp.float32)
                     for w in xew]
            for e, xec in ((2 * i, xe_lo), (2 * i + 1, xe_hi)):
                xlc = [xl_v[e, pl.ds(c * 16, 16)] for c in range(8)]
                acc = z
                for c in range(8):
                    m = xlc[c] + xr_v[e, pl.ds(c * 16, 16)] + xec[c]
                    m = jnp.maximum(m, 0.2 * m)
                    acc = acc + m * attc[c]
                ex = jnp.exp(_lanesum(acc))
                out_v[e, pl.ds(ROW - 16, 16)] = ex
                for c in range(8):
                    out_v[e, pl.ds(c * 16, 16)] = xlc[c] * ex
            return ecarry

        lax.fori_loop(0, B // 2, ebody, 0)

    issue_gathers(0, 0)
    issue_gathers(1, 1)

    def grp_body(g, carry):
        for b in range(2):
            blk = 2 * g + b
            wait_gathers(b, blk)

            @pl.when(blk >= 2)
            def _():
                pltpu.make_async_copy(
                    out_vs[b], acc_sh.at[_idx(idxd_v, blk - 2)], ssems[b]
                ).wait()

            compute_block(b)
            pltpu.async_copy(out_vs[b], acc_sh.at[_idx(idxd_v, blk)], ssems[b],
                             add=True)

            @pl.when(blk + 2 < NBLK)
            def _():
                @pl.when((blk + 2) % IC == 0)
                def _():
                    wait_idx_chunk((blk + 2) // IC)

                issue_gathers(b, blk + 2)

            if b == 0:
                @pl.when(jnp.logical_and(blk % IC == 0, blk + IC < NBLK))
                def _():
                    issue_idx_chunk(blk // IC + 1)

        return carry

    lax.fori_loop(0, NBLK // 2, grp_body, 0)

    pltpu.make_async_copy(
        out_v0, acc_sh.at[_idx(idxd_v, NBLK - 2)], ssem0).wait()
    pltpu.make_async_copy(
        out_v1, acc_sh.at[_idx(idxd_v, NBLK - 1)], ssem1).wait()
    plsc.subcore_barrier()

    def exp_body(k, carry):
        j = sid + k * NS

        @pl.when(j < NROWCHUNK)
        def _():
            pltpu.sync_copy(acc_sh.at[pl.ds(j * B, B)], out_v0)
            pltpu.sync_copy(out_v0, acc_h.at[cid, pl.ds(j * B, B)])

        return carry

    lax.fori_loop(0, (NROWCHUNK + NS - 1) // NS, exp_body, 0)


_edge_pass = pl.kernel(
    _edge_pass_body,
    out_type=jax.ShapeDtypeStruct((NC, N, ROW), jnp.float32),
    mesh=plsc.VectorSubcoreMesh(core_axis_name="c", subcore_axis_name="s"),
    compiler_params=pltpu.CompilerParams(
        needs_layout_passes=False, use_tc_tiling_on_sc=False
    ),
    scratch_types=[
        pltpu.VMEM((2, IC, B), jnp.int32),
        pltpu.VMEM((2, IC, B), jnp.int32),
        pltpu.VMEM((B, D), jnp.float32),
        pltpu.VMEM((B, D), jnp.float32),
        pltpu.VMEM((B, D), jnp.float32),
        pltpu.VMEM((B, D), jnp.float32),
        pltpu.VMEM((B // 2, D), jnp.int32),
        pltpu.VMEM((B // 2, D), jnp.int32),
        pltpu.VMEM((B, ROW), jnp.float32),
        pltpu.VMEM((B, ROW), jnp.float32),
        pltpu.VMEM((D,), jnp.float32),
        pltpu.VMEM_SHARED((N, ROW), jnp.float32),
        pltpu.SemaphoreType.DMA,
        pltpu.SemaphoreType.DMA,
        pltpu.SemaphoreType.DMA,
        pltpu.SemaphoreType.DMA,
        pltpu.SemaphoreType.DMA,
    ],
)


def kernel(x, edge_index, edge_attr, Wl1, Wr1, We1, att1, b1, Wl2, Wr2, We2, att2, b2):
    src = edge_index[0].reshape(NW, NBLK, B)
    dst = edge_index[1].reshape(NW, NBLK, B)
    xl1, xr1 = _node_mm(x, Wl1, Wr1)
    xe1 = _edge_mm(edge_attr, We1)
    acc1 = _edge_pass(src, dst, xl1, xr1, xe1, att1.reshape(D))
    xe2 = _edge_mm(edge_attr, We2)
    xl2, xr2 = _combine_mm(acc1, b1.reshape(1, D), Wl2, Wr2)
    acc2 = _edge_pass(src, dst, xl2, xr2, xe2, att2.reshape(D))
    return _combine_final(acc2, b2.reshape(1, D))
